# baseline (device time: 14731 ns/iter reference)
import jax
import jax.numpy as jnp
from jax import lax
from jax.experimental import pallas as pl
from jax.experimental.pallas import tpu as pltpu

K = 8
N_DEV = 8
ROWS = 256
SUB = 256


def _bits(t):
    return ((t >> 2) & 1, (t >> 1) & 1, t & 1)


def _topk_desc(data, k):
    neg_inf = jnp.array(-jnp.inf, data.dtype)
    cols = []
    for _ in range(k):
        m = jnp.max(data, axis=1, keepdims=True)
        cols.append(m)
        data = jnp.where(data == m, neg_inf, data)
    return jnp.concatenate(cols, axis=1)


def kernel(x):
    rows, _ = x.shape

    def body(x_ref, out_ref, cand_ref, gather_ref, send_sems, recv_sems):
        my_x = lax.axis_index("x")
        my_y = lax.axis_index("y")
        my_z = lax.axis_index("z")
        my_sid = 4 * my_x + 2 * my_y + my_z

        barrier_sem = pltpu.get_barrier_semaphore()
        for t in range(N_DEV):

            @pl.when(my_sid != t)
            def _(t=t):
                pl.semaphore_signal(
                    barrier_sem,
                    inc=1,
                    device_id=_bits(t),
                    device_id_type=pl.DeviceIdType.MESH,
                )

        sub = 2 * my_x + my_z
        cand_ref[:, :] = _topk_desc(x_ref[:, pl.ds(sub * SUB, SUB)], K)

        pl.semaphore_wait(barrier_sem, N_DEV - 1)

        for s in range(N_DEV):

            @pl.when(my_sid == s)
            def _(s=s):
                gather_ref[s] = cand_ref[:, :]
                for t in range(N_DEV):
                    if t == s:
                        continue
                    rdma = pltpu.make_async_remote_copy(
                        src_ref=cand_ref,
                        dst_ref=gather_ref.at[s],
                        send_sem=send_sems.at[t],
                        recv_sem=recv_sems.at[s],
                        device_id=_bits(t),
                        device_id_type=pl.DeviceIdType.MESH,
                    )
                    rdma.start()

        for s in range(N_DEV):

            @pl.when(my_sid != s)
            def _(s=s):
                recv = pltpu.make_async_remote_copy(
                    src_ref=cand_ref,
                    dst_ref=gather_ref.at[s],
                    send_sem=send_sems.at[s],
                    recv_sem=recv_sems.at[s],
                    device_id=_bits(s),
                    device_id_type=pl.DeviceIdType.MESH,
                )
                recv.wait_recv()

        both = jnp.concatenate([gather_ref[s] for s in range(N_DEV)], axis=1)
        out_ref[:, :] = _topk_desc(both, K)

        for t in range(N_DEV):

            @pl.when(my_sid != t)
            def _(t=t):
                send = pltpu.make_async_remote_copy(
                    src_ref=cand_ref,
                    dst_ref=gather_ref.at[t],
                    send_sem=send_sems.at[t],
                    recv_sem=recv_sems.at[t],
                    device_id=_bits(t),
                    device_id_type=pl.DeviceIdType.MESH,
                )
                send.wait_send()

    return pl.pallas_call(
        body,
        out_shape=jax.ShapeDtypeStruct((rows, K), jnp.float32),
        in_specs=[pl.BlockSpec(memory_space=pltpu.VMEM)],
        out_specs=pl.BlockSpec(memory_space=pltpu.VMEM),
        scratch_shapes=[
            pltpu.VMEM((rows, K), jnp.float32),
            pltpu.VMEM((N_DEV, rows, K), jnp.float32),
            pltpu.SemaphoreType.DMA((N_DEV,)),
            pltpu.SemaphoreType.DMA((N_DEV,)),
        ],
        compiler_params=pltpu.CompilerParams(collective_id=0),
    )(x)
